# Initial kernel scaffold; baseline (speedup 1.0000x reference)
import jax
import jax.numpy as jnp
from jax.experimental import pallas as pl


def kernel(x, edge_index, Wl1, bl1, Wr1, br1, att1, bias1, g1, b1, m1, v1, Wl2, bl2, Wr2, br2, att2, bias2, g2, b2, m2, v2, Wc, bc):
    raise NotImplementedError("write your pallas kernel here")


# speedup vs baseline: 9.5299x; 9.5299x over previous
import functools

import jax
import jax.numpy as jnp
from jax import lax
from jax.experimental import pallas as pl
from jax.experimental.pallas import tpu as pltpu
from jax.experimental.pallas import tpu_sc as plsc

N = 10000
D_IN = 128
HID = 64
HEADS = 2
F1 = HEADS * HID
F2 = HID
E = 320000
ET = E + N
NC, NS, LANES = 2, 16, 16
NW = NC * NS
K = 128
NCH = 81
EP = NCH * K
E_PAD = NW * EP
CW1 = 144
CW2 = 80
NPAD = 10240
RPT = NPAD // NS
FLUSH_R = 128
NEG = -1e30
BN_EPS = 1e-5
BR = 1000


def _mesh():
    return plsc.VectorSubcoreMesh(
        core_axis_name="c", subcore_axis_name="s", num_cores=NC, num_subcores=NS)


def _make_alpha(F, nh):
    chans = F // nh
    out_type = tuple(
        [jax.ShapeDtypeStruct((E_PAD,), jnp.float32) for _ in range(nh)]
        + [jax.ShapeDtypeStruct((NW * nh * LANES,), jnp.float32)])
    scratch = [
        pltpu.VMEM((K,), jnp.int32),
        pltpu.VMEM((K,), jnp.int32),
        pltpu.VMEM((K, F), jnp.float32),
        pltpu.VMEM((K, F), jnp.float32),
        pltpu.VMEM((nh * K,), jnp.float32),
        pltpu.VMEM((F + 8,), jnp.float32),
        pltpu.VMEM((nh * LANES,), jnp.float32),
        pltpu.SemaphoreType.DMA,
        pltpu.SemaphoreType.DMA,
    ]

    @functools.partial(pl.kernel, out_type=out_type, mesh=_mesh(),
                       compiler_params=pltpu.CompilerParams(
                           needs_layout_passes=False,
                           use_tc_tiling_on_sc=False),
                       scratch_types=scratch)
    def alpha_kernel(src_h, dst_h, xl_h, xr_h, att_h, *rest):
        a_hs = rest[:nh]
        tmax_h = rest[nh]
        (src_v, dst_v, rows_a, rows_b, albuf, att_v, mx_v, sem_a,
         sem_b) = rest[nh + 1:]
        cid = lax.axis_index("c")
        sid = lax.axis_index("s")
        wid = sid * NC + cid
        lanes = lax.iota(jnp.int32, LANES)
        pltpu.sync_copy(att_h, att_v.at[pl.ds(8, F)])
        for h in range(nh):
            mx_v[pl.ds(h * LANES, LANES)] = jnp.full((LANES,), NEG, jnp.float32)

        def chunk(i, carry):
            base = wid * EP + i * K
            pltpu.sync_copy(src_h.at[pl.ds(base, K)], src_v)
            pltpu.sync_copy(dst_h.at[pl.ds(base, K)], dst_v)
            cp_a = pltpu.async_copy(xl_h.at[src_v], rows_a, sem_a)
            cp_b = pltpu.async_copy(xr_h.at[dst_v], rows_b, sem_b)
            cp_a.wait()
            cp_b.wait()

            def group(g, gcarry):
                elanes = lanes + g * LANES
                accs = []
                for h in range(nh):
                    acc = jnp.zeros((LANES,), jnp.float32)
                    for cc in range(chans):
                        c = h * chans + cc
                        colv = jnp.full((LANES,), c, jnp.int32)
                        a = plsc.load_gather(rows_a, [elanes, colv])
                        b = plsc.load_gather(rows_b, [elanes, colv])
                        s = a + b
                        l = jnp.maximum(s, 0.2 * s)
                        w = plsc.load_gather(
                            att_v, [jnp.full((LANES,), c + 8, jnp.int32)])
                        acc = acc + l * w
                    accs.append(acc)
                valid = (elanes + base) < ET
                for h in range(nh):
                    am = jnp.where(valid, accs[h], NEG)
                    albuf[pl.ds(h * K + g * LANES, LANES)] = am
                    mh = mx_v[pl.ds(h * LANES, LANES)]
                    mx_v[pl.ds(h * LANES, LANES)] = jnp.maximum(mh, am)
                return gcarry

            lax.fori_loop(0, K // LANES, group, 0)
            for h in range(nh):
                pltpu.sync_copy(albuf.at[pl.ds(h * K, K)],
                                a_hs[h].at[pl.ds(base, K)])
            return carry

        lax.fori_loop(0, NCH, chunk, 0)
        pltpu.sync_copy(mx_v, tmax_h.at[pl.ds(wid * nh * LANES, nh * LANES)])

    return alpha_kernel


def _make_agg(F, nh, cw):
    chans = F // nh
    out_type = jax.ShapeDtypeStruct((NC, NPAD, cw), jnp.float32)
    scratch = [
        pltpu.VMEM((K,), jnp.int32),
        pltpu.VMEM((K,), jnp.int32),
        pltpu.VMEM((nh * K,), jnp.float32),
        pltpu.VMEM((K, F), jnp.float32),
        pltpu.VMEM((K, cw), jnp.float32),
        pltpu.VMEM((NW * nh * LANES,), jnp.float32),
        pltpu.VMEM_SHARED((NPAD, cw), jnp.float32),
        pltpu.SemaphoreType.DMA,
    ]

    @functools.partial(pl.kernel, out_type=out_type, mesh=_mesh(),
                       compiler_params=pltpu.CompilerParams(
                           needs_layout_passes=False,
                           use_tc_tiling_on_sc=False),
                       scratch_types=scratch)
    def agg_kernel(src_h, dst_h, xl_h, tmax_h, *rest):
        a_hs = rest[:nh]
        out_h = rest[nh]
        (src_v, dst_v, a_v, rows_a, msg, tm_v, acc_sh, sem_a) = rest[nh + 1:]
        cid = lax.axis_index("c")
        sid = lax.axis_index("s")
        wid = sid * NC + cid
        lanes = lax.iota(jnp.int32, LANES)

        pltpu.sync_copy(tmax_h, tm_v)
        gmax = []
        for h in range(nh):
            def red(w, m, h=h):
                return jnp.maximum(
                    m, tm_v[pl.ds(w * nh * LANES + h * LANES, LANES)])
            mh = lax.fori_loop(0, NW, red, jnp.full((LANES,), NEG, jnp.float32))
            gmax.append(jnp.max(mh))

        def zrow(r, carry):
            for cc in range(cw // LANES):
                plsc.store_scatter(
                    msg, [jnp.full((LANES,), r, jnp.int32), lanes + cc * LANES],
                    jnp.zeros((LANES,), jnp.float32))
            return carry
        lax.fori_loop(0, K, zrow, 0)

        r0 = sid * RPT
        for j in range(RPT // FLUSH_R):
            pltpu.sync_copy(msg.at[pl.ds(0, FLUSH_R)],
                            acc_sh.at[pl.ds(r0 + j * FLUSH_R, FLUSH_R)])
        plsc.subcore_barrier()

        def chunk(i, carry):
            base = wid * EP + i * K
            pltpu.sync_copy(src_h.at[pl.ds(base, K)], src_v)
            pltpu.sync_copy(dst_h.at[pl.ds(base, K)], dst_v)
            for h in range(nh):
                pltpu.sync_copy(a_hs[h].at[pl.ds(base, K)],
                                a_v.at[pl.ds(h * K, K)])
            cp_a = pltpu.async_copy(xl_h.at[src_v], rows_a, sem_a)
            cp_a.wait()

            def group(g, gcarry):
                elanes = lanes + g * LANES
                evs = []
                for h in range(nh):
                    av = a_v[pl.ds(h * K + g * LANES, LANES)]
                    evs.append(jnp.exp(av - gmax[h]))
                for h in range(nh):
                    plsc.store_scatter(
                        msg, [elanes, jnp.full((LANES,), F + h, jnp.int32)],
                        evs[h])
                for h in range(nh):
                    for cc in range(chans):
                        c = h * chans + cc
                        colv = jnp.full((LANES,), c, jnp.int32)
                        v = plsc.load_gather(rows_a, [elanes, colv])
                        plsc.store_scatter(msg, [elanes, colv], v * evs[h])
                return gcarry

            lax.fori_loop(0, K // LANES, group, 0)
            pltpu.sync_copy(msg, acc_sh.at[dst_v], add=True)
            return carry

        lax.fori_loop(0, NCH, chunk, 0)
        plsc.subcore_barrier()

        for j in range(RPT // FLUSH_R):
            rr = r0 + j * FLUSH_R
            pltpu.sync_copy(acc_sh.at[pl.ds(rr, FLUSH_R)],
                            msg.at[pl.ds(0, FLUSH_R)])
            pltpu.sync_copy(msg.at[pl.ds(0, FLUSH_R)],
                            out_h.at[cid, pl.ds(rr, FLUSH_R)])

    return agg_kernel


_alpha1 = _make_alpha(F1, HEADS)
_alpha2 = _make_alpha(F2, 1)
_agg1 = _make_agg(F1, HEADS, CW1)
_agg2 = _make_agg(F2, 1, CW2)


def _proj_body(x_ref, wl_ref, bl_ref, wr_ref, br_ref, xl_ref, xr_ref):
    xx = x_ref[...]
    xl_ref[...] = jnp.dot(xx, wl_ref[...],
                          preferred_element_type=jnp.float32) + bl_ref[...]
    xr_ref[...] = jnp.dot(xx, wr_ref[...],
                          preferred_element_type=jnp.float32) + br_ref[...]


def _tc_proj(x, wl, bl, wr, br):
    d = x.shape[1]
    f = wl.shape[1]
    return pl.pallas_call(
        _proj_body,
        grid=(N // BR,),
        in_specs=[
            pl.BlockSpec((BR, d), lambda i: (i, 0)),
            pl.BlockSpec((d, f), lambda i: (0, 0)),
            pl.BlockSpec((1, f), lambda i: (0, 0)),
            pl.BlockSpec((d, f), lambda i: (0, 0)),
            pl.BlockSpec((1, f), lambda i: (0, 0)),
        ],
        out_specs=[pl.BlockSpec((BR, f), lambda i: (i, 0)),
                   pl.BlockSpec((BR, f), lambda i: (i, 0))],
        out_shape=[jax.ShapeDtypeStruct((N, f), jnp.float32),
                   jax.ShapeDtypeStruct((N, f), jnp.float32)],
    )(x, wl, bl.reshape(1, f), wr, br.reshape(1, f))


def _mid_body(acc_ref, bias_ref, g_ref, b_ref, m_ref, v_ref, wl_ref, bl_ref,
              wr_ref, br_ref, xl_ref, xr_ref):
    s = acc_ref[0] + acc_ref[1]
    h0 = s[:, 0:HID] / (s[:, F1:F1 + 1] + 1e-16)
    h1 = s[:, HID:F1] / (s[:, F1 + 1:F1 + 2] + 1e-16)
    h = jnp.concatenate([h0, h1], axis=1) + bias_ref[...]
    h = (h - m_ref[...]) * (g_ref[...] * lax.rsqrt(v_ref[...] + BN_EPS)) \
        + b_ref[...]
    h = jnp.where(h > 0, h, jnp.exp(h) - 1.0)
    xl_ref[...] = jnp.dot(h, wl_ref[...],
                          preferred_element_type=jnp.float32) + bl_ref[...]
    xr_ref[...] = jnp.dot(h, wr_ref[...],
                          preferred_element_type=jnp.float32) + br_ref[...]


def _tc_mid(acc, bias1, g1, b1, m1, v1, wl2, bl2, wr2, br2):
    vec = lambda a: a.reshape(1, F1)
    return pl.pallas_call(
        _mid_body,
        grid=(N // BR,),
        in_specs=[
            pl.BlockSpec((NC, BR, CW1), lambda i: (0, i, 0)),
            pl.BlockSpec((1, F1), lambda i: (0, 0)),
            pl.BlockSpec((1, F1), lambda i: (0, 0)),
            pl.BlockSpec((1, F1), lambda i: (0, 0)),
            pl.BlockSpec((1, F1), lambda i: (0, 0)),
            pl.BlockSpec((1, F1), lambda i: (0, 0)),
            pl.BlockSpec((F1, F2), lambda i: (0, 0)),
            pl.BlockSpec((1, F2), lambda i: (0, 0)),
            pl.BlockSpec((F1, F2), lambda i: (0, 0)),
            pl.BlockSpec((1, F2), lambda i: (0, 0)),
        ],
        out_specs=[pl.BlockSpec((BR, F2), lambda i: (i, 0)),
                   pl.BlockSpec((BR, F2), lambda i: (i, 0))],
        out_shape=[jax.ShapeDtypeStruct((N, F2), jnp.float32),
                   jax.ShapeDtypeStruct((N, F2), jnp.float32)],
    )(acc, vec(bias1), vec(g1), vec(b1), vec(m1), vec(v1),
      wl2, bl2.reshape(1, F2), wr2, br2.reshape(1, F2))


def _final_body(acc_ref, bias_ref, g_ref, b_ref, m_ref, v_ref, wc_ref, bc_ref,
                o_ref):
    s = acc_ref[0] + acc_ref[1]
    h = s[:, 0:HID] / (s[:, HID:HID + 1] + 1e-16) + bias_ref[...]
    h = (h - m_ref[...]) * (g_ref[...] * lax.rsqrt(v_ref[...] + BN_EPS)) \
        + b_ref[...]
    h = jnp.where(h > 0, h, jnp.exp(h) - 1.0)
    o = jnp.dot(h, wc_ref[...], preferred_element_type=jnp.float32) \
        + bc_ref[...]
    o_ref[...] = 1.0 / (1.0 + jnp.exp(-o))


def _tc_final(acc, bias2, g2, b2, m2, v2, wc_pad, bc_pad):
    vec = lambda a: a.reshape(1, F2)
    return pl.pallas_call(
        _final_body,
        grid=(N // BR,),
        in_specs=[
            pl.BlockSpec((NC, BR, CW2), lambda i: (0, i, 0)),
            pl.BlockSpec((1, F2), lambda i: (0, 0)),
            pl.BlockSpec((1, F2), lambda i: (0, 0)),
            pl.BlockSpec((1, F2), lambda i: (0, 0)),
            pl.BlockSpec((1, F2), lambda i: (0, 0)),
            pl.BlockSpec((1, F2), lambda i: (0, 0)),
            pl.BlockSpec((F2, 128), lambda i: (0, 0)),
            pl.BlockSpec((1, 128), lambda i: (0, 0)),
        ],
        out_specs=pl.BlockSpec((BR, 128), lambda i: (i, 0)),
        out_shape=jax.ShapeDtypeStruct((N, 128), jnp.float32),
    )(acc, vec(bias2), vec(g2), vec(b2), vec(m2), vec(v2), wc_pad, bc_pad)


def kernel(x, edge_index, Wl1, bl1, Wr1, br1, att1, bias1, g1, b1, m1, v1,
           Wl2, bl2, Wr2, br2, att2, bias2, g2, b2, m2, v2, Wc, bc):
    loops = jnp.arange(N, dtype=jnp.int32)
    pad = jnp.zeros((E_PAD - ET,), jnp.int32)
    src = jnp.concatenate([edge_index[0].astype(jnp.int32), loops, pad])
    dst = jnp.concatenate([edge_index[1].astype(jnp.int32), loops, pad])

    xl1, xr1 = _tc_proj(x, Wl1, bl1, Wr1, br1)
    a0, a1, tmax1 = _alpha1(src, dst, xl1, xr1, att1.reshape(F1))
    acc1 = _agg1(src, dst, xl1, tmax1, a0, a1)[:, :N, :]
    xl2, xr2 = _tc_mid(acc1, bias1, g1, b1, m1, v1, Wl2, bl2, Wr2, br2)
    b0, tmax2 = _alpha2(src, dst, xl2, xr2, att2.reshape(F2))
    acc2 = _agg2(src, dst, xl2, tmax2, b0)[:, :N, :]
    wc_pad = jnp.pad(Wc, ((0, 0), (0, 127)))
    bc_pad = jnp.pad(bc, (0, 127)).reshape(1, 128)
    out = _tc_final(acc2, bias2, g2, b2, m2, v2, wc_pad, bc_pad)
    return out[:, :1]


# speedup vs baseline: 11.5571x; 1.2127x over previous
import functools

import jax
import jax.numpy as jnp
from jax import lax
from jax.experimental import pallas as pl
from jax.experimental.pallas import tpu as pltpu
from jax.experimental.pallas import tpu_sc as plsc

N = 10000
D_IN = 128
HID = 64
HEADS = 2
F1 = HEADS * HID
F2 = HID
E = 320000
ET = E + N
NC, NS, LANES = 2, 16, 16
NW = NC * NS
K = 128
NCH = 84
EP = NCH * K
E_PAD = NW * EP
CW1 = 144
CW2 = 80
NPAD = 10240
RPT = NPAD // NS
FLUSH_R = 128
NEG = -1e30
BN_EPS = 1e-5
BR = 1000

_SC_PARAMS = pltpu.CompilerParams(
    needs_layout_passes=False, use_tc_tiling_on_sc=False)


def _mesh():
    return plsc.VectorSubcoreMesh(
        core_axis_name="c", subcore_axis_name="s", num_cores=NC, num_subcores=NS)


def _make_alpha(F, nh):
    chans = F // nh
    out_type = tuple(
        [jax.ShapeDtypeStruct((E_PAD,), jnp.float32) for _ in range(nh)]
        + [jax.ShapeDtypeStruct((NW * nh * LANES,), jnp.float32)])
    scratch = (
        [pltpu.VMEM((K,), jnp.int32)] * 4
        + [pltpu.VMEM((K, F), jnp.float32)] * 4
        + [pltpu.VMEM((nh * K,), jnp.float32),
           pltpu.VMEM((F + 8,), jnp.float32),
           pltpu.VMEM((nh * LANES,), jnp.float32)]
        + [pltpu.SemaphoreType.DMA] * 6)

    @functools.partial(pl.kernel, out_type=out_type, mesh=_mesh(),
                       compiler_params=_SC_PARAMS, scratch_types=scratch)
    def alpha_kernel(src_h, dst_h, xl_h, xr_h, att_h, *rest):
        a_hs = rest[:nh]
        tmax_h = rest[nh]
        (s0, s1, d0, d1, ra0, ra1, rb0, rb1, albuf, att_v, mx_v,
         si0, si1, sa0, sa1, sb0, sb1) = rest[nh + 1:]
        src_v, dst_v = [s0, s1], [d0, d1]
        rows_a, rows_b = [ra0, ra1], [rb0, rb1]
        sem_i, sem_a, sem_b = [si0, si1], [sa0, sa1], [sb0, sb1]
        cid = lax.axis_index("c")
        sid = lax.axis_index("s")
        wid = sid * NC + cid
        lanes = lax.iota(jnp.int32, LANES)
        pltpu.sync_copy(att_h, att_v.at[pl.ds(8, F)])
        for h in range(nh):
            mx_v[pl.ds(h * LANES, LANES)] = jnp.full((LANES,), NEG, jnp.float32)

        def start_idx(c, s):
            base = wid * EP + c * K
            pltpu.async_copy(src_h.at[pl.ds(base, K)], src_v[s], sem_i[s])
            pltpu.async_copy(dst_h.at[pl.ds(base, K)], dst_v[s], sem_i[s])

        def wait_idx(s):
            pltpu.make_async_copy(src_h.at[pl.ds(0, K)], src_v[s],
                                  sem_i[s]).wait()
            pltpu.make_async_copy(dst_h.at[pl.ds(0, K)], dst_v[s],
                                  sem_i[s]).wait()

        def start_rows(s):
            pltpu.async_copy(xl_h.at[src_v[s]], rows_a[s], sem_a[s])
            pltpu.async_copy(xr_h.at[dst_v[s]], rows_b[s], sem_b[s])

        def wait_rows(s):
            pltpu.make_async_copy(xl_h.at[src_v[s]], rows_a[s],
                                  sem_a[s]).wait()
            pltpu.make_async_copy(xr_h.at[dst_v[s]], rows_b[s],
                                  sem_b[s]).wait()

        def compute(c, s):
            base = wid * EP + c * K
            ra, rb = rows_a[s], rows_b[s]

            def group(g, gcarry):
                elanes = lanes + g * LANES
                accs = []
                for h in range(nh):
                    acc = jnp.zeros((LANES,), jnp.float32)
                    for cc in range(chans):
                        ch = h * chans + cc
                        colv = jnp.full((LANES,), ch, jnp.int32)
                        a = plsc.load_gather(ra, [elanes, colv])
                        b = plsc.load_gather(rb, [elanes, colv])
                        ss = a + b
                        l = jnp.maximum(ss, 0.2 * ss)
                        w = plsc.load_gather(
                            att_v, [jnp.full((LANES,), ch + 8, jnp.int32)])
                        acc = acc + l * w
                    accs.append(acc)
                valid = (elanes + base) < ET
                for h in range(nh):
                    am = jnp.where(valid, accs[h], NEG)
                    albuf[pl.ds(h * K + g * LANES, LANES)] = am
                    mh = mx_v[pl.ds(h * LANES, LANES)]
                    mx_v[pl.ds(h * LANES, LANES)] = jnp.maximum(mh, am)
                return gcarry

            lax.fori_loop(0, K // LANES, group, 0)
            for h in range(nh):
                pltpu.sync_copy(albuf.at[pl.ds(h * K, K)],
                                a_hs[h].at[pl.ds(base, K)])

        start_idx(0, 0)
        start_idx(1, 1)
        wait_idx(0)
        start_rows(0)
        wait_idx(1)
        start_rows(1)

        def body(i2, carry):
            for s in range(2):
                c = 2 * i2 + s
                wait_rows(s)

                @pl.when(c + 2 < NCH)
                def _(s=s, c=c):
                    start_idx(c + 2, s)

                compute(c, s)

                @pl.when(c + 2 < NCH)
                def _(s=s):
                    wait_idx(s)
                    start_rows(s)
            return carry

        lax.fori_loop(0, NCH // 2, body, 0)
        pltpu.sync_copy(mx_v, tmax_h.at[pl.ds(wid * nh * LANES, nh * LANES)])

    return alpha_kernel


def _make_agg(F, nh, cw, kk):
    chans = F // nh
    nch = EP // kk
    assert nch % 4 == 0
    out_type = jax.ShapeDtypeStruct((NC, NPAD, cw), jnp.float32)
    scratch = (
        [pltpu.VMEM((kk,), jnp.int32)] * 2
        + [pltpu.VMEM((kk,), jnp.int32)] * 4
        + [pltpu.VMEM((nh * kk,), jnp.float32)] * 2
        + [pltpu.VMEM((kk, F), jnp.float32)] * 2
        + [pltpu.VMEM((kk, cw), jnp.float32)] * 2
        + [pltpu.VMEM((NW * nh * LANES,), jnp.float32),
           pltpu.VMEM_SHARED((NPAD, cw), jnp.float32)]
        + [pltpu.SemaphoreType.DMA] * 8)

    @functools.partial(pl.kernel, out_type=out_type, mesh=_mesh(),
                       compiler_params=_SC_PARAMS, scratch_types=scratch)
    def agg_kernel(src_h, dst_h, xl_h, tmax_h, *rest):
        a_hs = rest[:nh]
        out_h = rest[nh]
        (s0, s1, q0, q1, q2, q3, a0, a1, r0, r1, m0, m1, tm_v, acc_sh,
         si0, si1, sl0, sl1, sr0, sr1, sc0, sc1) = rest[nh + 1:]
        src_v, dsc_v = [s0, s1], [q0, q1, q2, q3]
        a_v, rows_a, msg = [a0, a1], [r0, r1], [m0, m1]
        sem_i, sem_al = [si0, si1], [sl0, sl1]
        sem_r, sem_sc = [sr0, sr1], [sc0, sc1]
        cid = lax.axis_index("c")
        sid = lax.axis_index("s")
        wid = sid * NC + cid
        lanes = lax.iota(jnp.int32, LANES)

        pltpu.sync_copy(tmax_h, tm_v)
        gmax = []
        for h in range(nh):
            def red(w, m, h=h):
                return jnp.maximum(
                    m, tm_v[pl.ds(w * nh * LANES + h * LANES, LANES)])
            mh = lax.fori_loop(0, NW, red, jnp.full((LANES,), NEG, jnp.float32))
            gmax.append(jnp.max(mh))

        for s in range(2):
            def zrow(r, carry, s=s):
                for ccg in range(cw // LANES):
                    plsc.store_scatter(
                        msg[s],
                        [jnp.full((LANES,), r, jnp.int32), lanes + ccg * LANES],
                        jnp.zeros((LANES,), jnp.float32))
                return carry
            lax.fori_loop(0, kk, zrow, 0)

        tr0 = sid * RPT
        for j in range(RPT // kk):
            pltpu.sync_copy(msg[0].at[pl.ds(0, kk)],
                            acc_sh.at[pl.ds(tr0 + j * kk, kk)])
        plsc.subcore_barrier()

        def start_idx(c, s, d):
            base = wid * EP + c * kk
            pltpu.async_copy(src_h.at[pl.ds(base, kk)], src_v[s], sem_i[s])
            pltpu.async_copy(dst_h.at[pl.ds(base, kk)], dsc_v[d], sem_i[s])

        def wait_idx(s, d):
            pltpu.make_async_copy(src_h.at[pl.ds(0, kk)], src_v[s],
                                  sem_i[s]).wait()
            pltpu.make_async_copy(dst_h.at[pl.ds(0, kk)], dsc_v[d],
                                  sem_i[s]).wait()

        def start_alpha(c, s):
            base = wid * EP + c * kk
            for h in range(nh):
                pltpu.async_copy(a_hs[h].at[pl.ds(base, kk)],
                                 a_v[s].at[pl.ds(h * kk, kk)], sem_al[s])

        def wait_alpha(s):
            for h in range(nh):
                pltpu.make_async_copy(a_hs[h].at[pl.ds(0, kk)],
                                      a_v[s].at[pl.ds(h * kk, kk)],
                                      sem_al[s]).wait()

        def start_rows(s):
            pltpu.async_copy(xl_h.at[src_v[s]], rows_a[s], sem_r[s])

        def wait_rows(s):
            pltpu.make_async_copy(xl_h.at[src_v[s]], rows_a[s],
                                  sem_r[s]).wait()

        def start_scatter(s, d):
            pltpu.async_copy(msg[s], acc_sh.at[dsc_v[d]], sem_sc[s], add=True)

        def wait_scatter(s, d):
            pltpu.make_async_copy(msg[s], acc_sh.at[dsc_v[d]],
                                  sem_sc[s]).wait()

        def compute(s):
            ra, mg, av = rows_a[s], msg[s], a_v[s]

            def group(g, gcarry):
                elanes = lanes + g * LANES
                evs = []
                for h in range(nh):
                    avv = av[pl.ds(h * kk + g * LANES, LANES)]
                    evs.append(jnp.exp(avv - gmax[h]))
                for h in range(nh):
                    plsc.store_scatter(
                        mg, [elanes, jnp.full((LANES,), F + h, jnp.int32)],
                        evs[h])
                for h in range(nh):
                    for cc in range(chans):
                        ch = h * chans + cc
                        colv = jnp.full((LANES,), ch, jnp.int32)
                        v = plsc.load_gather(ra, [elanes, colv])
                        plsc.store_scatter(mg, [elanes, colv], v * evs[h])
                return gcarry

            lax.fori_loop(0, kk // LANES, group, 0)

        start_idx(0, 0, 0)
        start_alpha(0, 0)
        start_idx(1, 1, 1)
        start_alpha(1, 1)
        wait_idx(0, 0)
        start_rows(0)
        wait_idx(1, 1)
        start_rows(1)

        def body(i2, carry):
            for j in range(4):
                sj, dj, dj2 = j % 2, j, (j + 2) % 4
                c = 4 * i2 + j
                wait_rows(sj)
                if j < 2:
                    @pl.when(i2 > 0)
                    def _(sj=sj, dj=dj):
                        wait_scatter(sj, dj)
                else:
                    wait_scatter(sj, j - 2)
                @pl.when(c + 2 < nch)
                def _(c=c, sj=sj, dj2=dj2):
                    start_idx(c + 2, sj, dj2)
                wait_alpha(sj)
                compute(sj)
                start_scatter(sj, dj)
                @pl.when(c + 2 < nch)
                def _(c=c, sj=sj):
                    start_alpha(c + 2, sj)
                    wait_idx(sj, (j + 2) % 4)
                    start_rows(sj)
            return carry

        lax.fori_loop(0, nch // 4, body, 0)
        wait_scatter(0, 2)
        wait_scatter(1, 3)
        plsc.subcore_barrier()

        for j in range(RPT // kk):
            rr = tr0 + j * kk
            pltpu.sync_copy(acc_sh.at[pl.ds(rr, kk)],
                            msg[0].at[pl.ds(0, kk)])
            pltpu.sync_copy(msg[0].at[pl.ds(0, kk)],
                            out_h.at[cid, pl.ds(rr, kk)])

    return agg_kernel


_alpha1 = _make_alpha(F1, HEADS)
_alpha2 = _make_alpha(F2, 1)
_agg1 = _make_agg(F1, HEADS, CW1, 64)
_agg2 = _make_agg(F2, 1, CW2, 128)


def _proj_body(x_ref, wl_ref, bl_ref, wr_ref, br_ref, xl_ref, xr_ref):
    xx = x_ref[...]
    xl_ref[...] = jnp.dot(xx, wl_ref[...],
                          preferred_element_type=jnp.float32) + bl_ref[...]
    xr_ref[...] = jnp.dot(xx, wr_ref[...],
                          preferred_element_type=jnp.float32) + br_ref[...]


def _tc_proj(x, wl, bl, wr, br):
    d = x.shape[1]
    f = wl.shape[1]
    return pl.pallas_call(
        _proj_body,
        grid=(N // BR,),
        in_specs=[
            pl.BlockSpec((BR, d), lambda i: (i, 0)),
            pl.BlockSpec((d, f), lambda i: (0, 0)),
            pl.BlockSpec((1, f), lambda i: (0, 0)),
            pl.BlockSpec((d, f), lambda i: (0, 0)),
            pl.BlockSpec((1, f), lambda i: (0, 0)),
        ],
        out_specs=[pl.BlockSpec((BR, f), lambda i: (i, 0)),
                   pl.BlockSpec((BR, f), lambda i: (i, 0))],
        out_shape=[jax.ShapeDtypeStruct((N, f), jnp.float32),
                   jax.ShapeDtypeStruct((N, f), jnp.float32)],
    )(x, wl, bl.reshape(1, f), wr, br.reshape(1, f))


def _mid_body(acc_ref, bias_ref, g_ref, b_ref, m_ref, v_ref, wl_ref, bl_ref,
              wr_ref, br_ref, xl_ref, xr_ref):
    s = acc_ref[0] + acc_ref[1]
    h0 = s[:, 0:HID] / (s[:, F1:F1 + 1] + 1e-16)
    h1 = s[:, HID:F1] / (s[:, F1 + 1:F1 + 2] + 1e-16)
    h = jnp.concatenate([h0, h1], axis=1) + bias_ref[...]
    h = (h - m_ref[...]) * (g_ref[...] * lax.rsqrt(v_ref[...] + BN_EPS)) \
        + b_ref[...]
    h = jnp.where(h > 0, h, jnp.exp(h) - 1.0)
    xl_ref[...] = jnp.dot(h, wl_ref[...],
                          preferred_element_type=jnp.float32) + bl_ref[...]
    xr_ref[...] = jnp.dot(h, wr_ref[...],
                          preferred_element_type=jnp.float32) + br_ref[...]


def _tc_mid(acc, bias1, g1, b1, m1, v1, wl2, bl2, wr2, br2):
    vec = lambda a: a.reshape(1, F1)
    return pl.pallas_call(
        _mid_body,
        grid=(N // BR,),
        in_specs=[
            pl.BlockSpec((NC, BR, CW1), lambda i: (0, i, 0)),
            pl.BlockSpec((1, F1), lambda i: (0, 0)),
            pl.BlockSpec((1, F1), lambda i: (0, 0)),
            pl.BlockSpec((1, F1), lambda i: (0, 0)),
            pl.BlockSpec((1, F1), lambda i: (0, 0)),
            pl.BlockSpec((1, F1), lambda i: (0, 0)),
            pl.BlockSpec((F1, F2), lambda i: (0, 0)),
            pl.BlockSpec((1, F2), lambda i: (0, 0)),
            pl.BlockSpec((F1, F2), lambda i: (0, 0)),
            pl.BlockSpec((1, F2), lambda i: (0, 0)),
        ],
        out_specs=[pl.BlockSpec((BR, F2), lambda i: (i, 0)),
                   pl.BlockSpec((BR, F2), lambda i: (i, 0))],
        out_shape=[jax.ShapeDtypeStruct((N, F2), jnp.float32),
                   jax.ShapeDtypeStruct((N, F2), jnp.float32)],
    )(acc, vec(bias1), vec(g1), vec(b1), vec(m1), vec(v1),
      wl2, bl2.reshape(1, F2), wr2, br2.reshape(1, F2))


def _final_body(acc_ref, bias_ref, g_ref, b_ref, m_ref, v_ref, wc_ref, bc_ref,
                o_ref):
    s = acc_ref[0] + acc_ref[1]
    h = s[:, 0:HID] / (s[:, HID:HID + 1] + 1e-16) + bias_ref[...]
    h = (h - m_ref[...]) * (g_ref[...] * lax.rsqrt(v_ref[...] + BN_EPS)) \
        + b_ref[...]
    h = jnp.where(h > 0, h, jnp.exp(h) - 1.0)
    o = jnp.dot(h, wc_ref[...], preferred_element_type=jnp.float32) \
        + bc_ref[...]
    o_ref[...] = 1.0 / (1.0 + jnp.exp(-o))


def _tc_final(acc, bias2, g2, b2, m2, v2, wc_pad, bc_pad):
    vec = lambda a: a.reshape(1, F2)
    return pl.pallas_call(
        _final_body,
        grid=(N // BR,),
        in_specs=[
            pl.BlockSpec((NC, BR, CW2), lambda i: (0, i, 0)),
            pl.BlockSpec((1, F2), lambda i: (0, 0)),
            pl.BlockSpec((1, F2), lambda i: (0, 0)),
            pl.BlockSpec((1, F2), lambda i: (0, 0)),
            pl.BlockSpec((1, F2), lambda i: (0, 0)),
            pl.BlockSpec((1, F2), lambda i: (0, 0)),
            pl.BlockSpec((F2, 128), lambda i: (0, 0)),
            pl.BlockSpec((1, 128), lambda i: (0, 0)),
        ],
        out_specs=pl.BlockSpec((BR, 128), lambda i: (i, 0)),
        out_shape=jax.ShapeDtypeStruct((N, 128), jnp.float32),
    )(acc, vec(bias2), vec(g2), vec(b2), vec(m2), vec(v2), wc_pad, bc_pad)


def kernel(x, edge_index, Wl1, bl1, Wr1, br1, att1, bias1, g1, b1, m1, v1,
           Wl2, bl2, Wr2, br2, att2, bias2, g2, b2, m2, v2, Wc, bc):
    loops = jnp.arange(N, dtype=jnp.int32)
    pad = jnp.zeros((E_PAD - ET,), jnp.int32)
    src = jnp.concatenate([edge_index[0].astype(jnp.int32), loops, pad])
    dst = jnp.concatenate([edge_index[1].astype(jnp.int32), loops, pad])

    xl1, xr1 = _tc_proj(x, Wl1, bl1, Wr1, br1)
    a0, a1, tmax1 = _alpha1(src, dst, xl1, xr1, att1.reshape(F1))
    acc1 = _agg1(src, dst, xl1, tmax1, a0, a1)[:, :N, :]
    xl2, xr2 = _tc_mid(acc1, bias1, g1, b1, m1, v1, Wl2, bl2, Wr2, br2)
    b0, tmax2 = _alpha2(src, dst, xl2, xr2, att2.reshape(F2))
    acc2 = _agg2(src, dst, xl2, tmax2, b0)[:, :N, :]
    wc_pad = jnp.pad(Wc, ((0, 0), (0, 127)))
    bc_pad = jnp.pad(bc, (0, 127)).reshape(1, 128)
    out = _tc_final(acc2, bias2, g2, b2, m2, v2, wc_pad, bc_pad)
    return out[:, :1]


# speedup vs baseline: 21.0419x; 1.8207x over previous
import functools

import jax
import jax.numpy as jnp
from jax import lax
from jax.experimental import pallas as pl
from jax.experimental.pallas import tpu as pltpu
from jax.experimental.pallas import tpu_sc as plsc

N = 10000
D_IN = 128
HID = 64
HEADS = 2
F1 = HEADS * HID
F2 = HID
E = 320000
ET = E + N
NC, NS, LANES = 2, 16, 16
NW = NC * NS
K = 128
NCH = 84
EP = NCH * K
E_PAD = NW * EP
CW1 = 144
CW2 = 80
NPAD = 10240
RPT = NPAD // NS
FLUSH_R = 128
NEG = -1e30
BN_EPS = 1e-5
BR = 1000

_SC_PARAMS = pltpu.CompilerParams(
    needs_layout_passes=False, use_tc_tiling_on_sc=False)


def _mesh():
    return plsc.VectorSubcoreMesh(
        core_axis_name="c", subcore_axis_name="s", num_cores=NC, num_subcores=NS)


def _make_alpha(F, nh):
    chans = F // nh
    out_type = tuple(
        [jax.ShapeDtypeStruct((E_PAD,), jnp.float32) for _ in range(nh)]
        + [jax.ShapeDtypeStruct((NW * nh * LANES,), jnp.float32)])
    scratch = (
        [pltpu.VMEM((K,), jnp.int32)] * 4
        + [pltpu.VMEM((K, F), jnp.float32)] * 4
        + [pltpu.VMEM((nh * K,), jnp.float32),
           pltpu.VMEM((F + 8,), jnp.float32),
           pltpu.VMEM((nh * LANES,), jnp.float32)]
        + [pltpu.SemaphoreType.DMA] * 6)

    @functools.partial(pl.kernel, out_type=out_type, mesh=_mesh(),
                       compiler_params=_SC_PARAMS, scratch_types=scratch)
    def alpha_kernel(src_h, dst_h, xl_h, xr_h, att_h, *rest):
        a_hs = rest[:nh]
        tmax_h = rest[nh]
        (s0, s1, d0, d1, ra0, ra1, rb0, rb1, albuf, att_v, mx_v,
         si0, si1, sa0, sa1, sb0, sb1) = rest[nh + 1:]
        src_v, dst_v = [s0, s1], [d0, d1]
        rows_a, rows_b = [ra0, ra1], [rb0, rb1]
        sem_i, sem_a, sem_b = [si0, si1], [sa0, sa1], [sb0, sb1]
        cid = lax.axis_index("c")
        sid = lax.axis_index("s")
        wid = sid * NC + cid
        lanes = lax.iota(jnp.int32, LANES)
        pltpu.sync_copy(att_h, att_v.at[pl.ds(8, F)])
        for h in range(nh):
            mx_v[pl.ds(h * LANES, LANES)] = jnp.full((LANES,), NEG, jnp.float32)

        def start_idx(c, s):
            base = wid * EP + c * K
            pltpu.async_copy(src_h.at[pl.ds(base, K)], src_v[s], sem_i[s])
            pltpu.async_copy(dst_h.at[pl.ds(base, K)], dst_v[s], sem_i[s])

        def wait_idx(s):
            pltpu.make_async_copy(src_h.at[pl.ds(0, K)], src_v[s],
                                  sem_i[s]).wait()
            pltpu.make_async_copy(dst_h.at[pl.ds(0, K)], dst_v[s],
                                  sem_i[s]).wait()

        def start_rows(s):
            pltpu.async_copy(xl_h.at[src_v[s]], rows_a[s], sem_a[s])
            pltpu.async_copy(xr_h.at[dst_v[s]], rows_b[s], sem_b[s])

        def wait_rows(s):
            pltpu.make_async_copy(xl_h.at[src_v[s]], rows_a[s],
                                  sem_a[s]).wait()
            pltpu.make_async_copy(xr_h.at[dst_v[s]], rows_b[s],
                                  sem_b[s]).wait()

        def compute(c, s):
            base = wid * EP + c * K
            ra, rb = rows_a[s], rows_b[s]
            att_vecs = [att_v[pl.ds(8 + cv * LANES, LANES)]
                        for cv in range(F // LANES)]
            cph = chans // LANES

            def group(g, gcarry):
                elanes = lanes + g * LANES
                als = [jnp.zeros((LANES,), jnp.float32) for _ in range(nh)]
                for e in range(LANES):
                    row = g * LANES + e
                    for h in range(nh):
                        sv = jnp.zeros((LANES,), jnp.float32)
                        for cv in range(cph):
                            off = h * chans + cv * LANES
                            a = ra[row, pl.ds(off, LANES)]
                            b = rb[row, pl.ds(off, LANES)]
                            ss = a + b
                            l = jnp.maximum(ss, 0.2 * ss)
                            sv = sv + l * att_vecs[h * cph + cv]
                        als[h] = jnp.where(lanes == e, jnp.sum(sv), als[h])
                valid = (elanes + base) < ET
                for h in range(nh):
                    am = jnp.where(valid, als[h], NEG)
                    albuf[pl.ds(h * K + g * LANES, LANES)] = am
                    mh = mx_v[pl.ds(h * LANES, LANES)]
                    mx_v[pl.ds(h * LANES, LANES)] = jnp.maximum(mh, am)
                return gcarry

            lax.fori_loop(0, K // LANES, group, 0)
            for h in range(nh):
                pltpu.sync_copy(albuf.at[pl.ds(h * K, K)],
                                a_hs[h].at[pl.ds(base, K)])

        start_idx(0, 0)
        start_idx(1, 1)
        wait_idx(0)
        start_rows(0)
        wait_idx(1)
        start_rows(1)

        def body(i2, carry):
            for s in range(2):
                c = 2 * i2 + s
                wait_rows(s)

                @pl.when(c + 2 < NCH)
                def _(s=s, c=c):
                    start_idx(c + 2, s)

                compute(c, s)

                @pl.when(c + 2 < NCH)
                def _(s=s):
                    wait_idx(s)
                    start_rows(s)
            return carry

        lax.fori_loop(0, NCH // 2, body, 0)
        pltpu.sync_copy(mx_v, tmax_h.at[pl.ds(wid * nh * LANES, nh * LANES)])

    return alpha_kernel


def _make_agg(F, nh, cw, kk):
    chans = F // nh
    nch = EP // kk
    assert nch % 4 == 0
    out_type = jax.ShapeDtypeStruct((NC, NPAD, cw), jnp.float32)
    scratch = (
        [pltpu.VMEM((kk,), jnp.int32)] * 2
        + [pltpu.VMEM((kk,), jnp.int32)] * 4
        + [pltpu.VMEM((nh * kk,), jnp.float32)] * 2
        + [pltpu.VMEM((kk, F), jnp.float32)] * 2
        + [pltpu.VMEM((kk, cw), jnp.float32)] * 2
        + [pltpu.VMEM((NW * nh * LANES,), jnp.float32),
           pltpu.VMEM_SHARED((NPAD, cw), jnp.float32)]
        + [pltpu.SemaphoreType.DMA] * 8)

    @functools.partial(pl.kernel, out_type=out_type, mesh=_mesh(),
                       compiler_params=_SC_PARAMS, scratch_types=scratch)
    def agg_kernel(src_h, dst_h, xl_h, tmax_h, *rest):
        a_hs = rest[:nh]
        out_h = rest[nh]
        (s0, s1, q0, q1, q2, q3, a0, a1, r0, r1, m0, m1, tm_v, acc_sh,
         si0, si1, sl0, sl1, sr0, sr1, sc0, sc1) = rest[nh + 1:]
        src_v, dsc_v = [s0, s1], [q0, q1, q2, q3]
        a_v, rows_a, msg = [a0, a1], [r0, r1], [m0, m1]
        sem_i, sem_al = [si0, si1], [sl0, sl1]
        sem_r, sem_sc = [sr0, sr1], [sc0, sc1]
        cid = lax.axis_index("c")
        sid = lax.axis_index("s")
        wid = sid * NC + cid
        lanes = lax.iota(jnp.int32, LANES)

        pltpu.sync_copy(tmax_h, tm_v)
        gmax = []
        for h in range(nh):
            def red(w, m, h=h):
                return jnp.maximum(
                    m, tm_v[pl.ds(w * nh * LANES + h * LANES, LANES)])
            mh = lax.fori_loop(0, NW, red, jnp.full((LANES,), NEG, jnp.float32))
            gmax.append(jnp.max(mh))

        for s in range(2):
            def zrow(r, carry, s=s):
                for ccg in range(cw // LANES):
                    plsc.store_scatter(
                        msg[s],
                        [jnp.full((LANES,), r, jnp.int32), lanes + ccg * LANES],
                        jnp.zeros((LANES,), jnp.float32))
                return carry
            lax.fori_loop(0, kk, zrow, 0)

        tr0 = sid * RPT
        for j in range(RPT // kk):
            pltpu.sync_copy(msg[0].at[pl.ds(0, kk)],
                            acc_sh.at[pl.ds(tr0 + j * kk, kk)])
        plsc.subcore_barrier()

        def start_idx(c, s, d):
            base = wid * EP + c * kk
            pltpu.async_copy(src_h.at[pl.ds(base, kk)], src_v[s], sem_i[s])
            pltpu.async_copy(dst_h.at[pl.ds(base, kk)], dsc_v[d], sem_i[s])

        def wait_idx(s, d):
            pltpu.make_async_copy(src_h.at[pl.ds(0, kk)], src_v[s],
                                  sem_i[s]).wait()
            pltpu.make_async_copy(dst_h.at[pl.ds(0, kk)], dsc_v[d],
                                  sem_i[s]).wait()

        def start_alpha(c, s):
            base = wid * EP + c * kk
            for h in range(nh):
                pltpu.async_copy(a_hs[h].at[pl.ds(base, kk)],
                                 a_v[s].at[pl.ds(h * kk, kk)], sem_al[s])

        def wait_alpha(s):
            for h in range(nh):
                pltpu.make_async_copy(a_hs[h].at[pl.ds(0, kk)],
                                      a_v[s].at[pl.ds(h * kk, kk)],
                                      sem_al[s]).wait()

        def start_rows(s):
            pltpu.async_copy(xl_h.at[src_v[s]], rows_a[s], sem_r[s])

        def wait_rows(s):
            pltpu.make_async_copy(xl_h.at[src_v[s]], rows_a[s],
                                  sem_r[s]).wait()

        def start_scatter(s, d):
            pltpu.async_copy(msg[s], acc_sh.at[dsc_v[d]], sem_sc[s], add=True)

        def wait_scatter(s, d):
            pltpu.make_async_copy(msg[s], acc_sh.at[dsc_v[d]],
                                  sem_sc[s]).wait()

        def compute(s):
            ra, mg, av = rows_a[s], msg[s], a_v[s]
            cph = chans // LANES
            zero16 = jnp.zeros((LANES,), jnp.float32)

            def group(g, gcarry):
                evs = []
                for h in range(nh):
                    avv = av[pl.ds(h * kk + g * LANES, LANES)]
                    evs.append(jnp.exp(avv - gmax[h]))
                for e in range(LANES):
                    row = g * LANES + e
                    sc = [jnp.sum(jnp.where(lanes == e, evs[h], 0.0))
                          for h in range(nh)]
                    for h in range(nh):
                        for cv in range(cph):
                            off = h * chans + cv * LANES
                            v = ra[row, pl.ds(off, LANES)]
                            mg[row, pl.ds(off, LANES)] = v * sc[h]
                    dv = zero16
                    for h in range(nh):
                        dv = jnp.where(lanes == h, sc[h], dv)
                    mg[row, pl.ds(F, LANES)] = dv
                return gcarry

            lax.fori_loop(0, kk // LANES, group, 0)

        start_idx(0, 0, 0)
        start_alpha(0, 0)
        start_idx(1, 1, 1)
        start_alpha(1, 1)
        wait_idx(0, 0)
        start_rows(0)
        wait_idx(1, 1)
        start_rows(1)

        def body(i2, carry):
            for j in range(4):
                sj, dj, dj2 = j % 2, j, (j + 2) % 4
                c = 4 * i2 + j
                wait_rows(sj)
                if j < 2:
                    @pl.when(i2 > 0)
                    def _(sj=sj, dj=dj):
                        wait_scatter(sj, dj)
                else:
                    wait_scatter(sj, j - 2)
                @pl.when(c + 2 < nch)
                def _(c=c, sj=sj, dj2=dj2):
                    start_idx(c + 2, sj, dj2)
                wait_alpha(sj)
                compute(sj)
                start_scatter(sj, dj)
                @pl.when(c + 2 < nch)
                def _(c=c, sj=sj):
                    start_alpha(c + 2, sj)
                    wait_idx(sj, (j + 2) % 4)
                    start_rows(sj)
            return carry

        lax.fori_loop(0, nch // 4, body, 0)
        wait_scatter(0, 2)
        wait_scatter(1, 3)
        plsc.subcore_barrier()

        for j in range(RPT // kk):
            rr = tr0 + j * kk
            pltpu.sync_copy(acc_sh.at[pl.ds(rr, kk)],
                            msg[0].at[pl.ds(0, kk)])
            pltpu.sync_copy(msg[0].at[pl.ds(0, kk)],
                            out_h.at[cid, pl.ds(rr, kk)])

    return agg_kernel


_alpha1 = _make_alpha(F1, HEADS)
_alpha2 = _make_alpha(F2, 1)
_agg1 = _make_agg(F1, HEADS, CW1, 64)
_agg2 = _make_agg(F2, 1, CW2, 128)


def _proj_body(x_ref, wl_ref, bl_ref, wr_ref, br_ref, xl_ref, xr_ref):
    xx = x_ref[...]
    xl_ref[...] = jnp.dot(xx, wl_ref[...],
                          preferred_element_type=jnp.float32) + bl_ref[...]
    xr_ref[...] = jnp.dot(xx, wr_ref[...],
                          preferred_element_type=jnp.float32) + br_ref[...]


def _tc_proj(x, wl, bl, wr, br):
    d = x.shape[1]
    f = wl.shape[1]
    return pl.pallas_call(
        _proj_body,
        grid=(N // BR,),
        in_specs=[
            pl.BlockSpec((BR, d), lambda i: (i, 0)),
            pl.BlockSpec((d, f), lambda i: (0, 0)),
            pl.BlockSpec((1, f), lambda i: (0, 0)),
            pl.BlockSpec((d, f), lambda i: (0, 0)),
            pl.BlockSpec((1, f), lambda i: (0, 0)),
        ],
        out_specs=[pl.BlockSpec((BR, f), lambda i: (i, 0)),
                   pl.BlockSpec((BR, f), lambda i: (i, 0))],
        out_shape=[jax.ShapeDtypeStruct((N, f), jnp.float32),
                   jax.ShapeDtypeStruct((N, f), jnp.float32)],
    )(x, wl, bl.reshape(1, f), wr, br.reshape(1, f))


def _mid_body(acc_ref, bias_ref, g_ref, b_ref, m_ref, v_ref, wl_ref, bl_ref,
              wr_ref, br_ref, xl_ref, xr_ref):
    s = acc_ref[0] + acc_ref[1]
    h0 = s[:, 0:HID] / (s[:, F1:F1 + 1] + 1e-16)
    h1 = s[:, HID:F1] / (s[:, F1 + 1:F1 + 2] + 1e-16)
    h = jnp.concatenate([h0, h1], axis=1) + bias_ref[...]
    h = (h - m_ref[...]) * (g_ref[...] * lax.rsqrt(v_ref[...] + BN_EPS)) \
        + b_ref[...]
    h = jnp.where(h > 0, h, jnp.exp(h) - 1.0)
    xl_ref[...] = jnp.dot(h, wl_ref[...],
                          preferred_element_type=jnp.float32) + bl_ref[...]
    xr_ref[...] = jnp.dot(h, wr_ref[...],
                          preferred_element_type=jnp.float32) + br_ref[...]


def _tc_mid(acc, bias1, g1, b1, m1, v1, wl2, bl2, wr2, br2):
    vec = lambda a: a.reshape(1, F1)
    return pl.pallas_call(
        _mid_body,
        grid=(N // BR,),
        in_specs=[
            pl.BlockSpec((NC, BR, CW1), lambda i: (0, i, 0)),
            pl.BlockSpec((1, F1), lambda i: (0, 0)),
            pl.BlockSpec((1, F1), lambda i: (0, 0)),
            pl.BlockSpec((1, F1), lambda i: (0, 0)),
            pl.BlockSpec((1, F1), lambda i: (0, 0)),
            pl.BlockSpec((1, F1), lambda i: (0, 0)),
            pl.BlockSpec((F1, F2), lambda i: (0, 0)),
            pl.BlockSpec((1, F2), lambda i: (0, 0)),
            pl.BlockSpec((F1, F2), lambda i: (0, 0)),
            pl.BlockSpec((1, F2), lambda i: (0, 0)),
        ],
        out_specs=[pl.BlockSpec((BR, F2), lambda i: (i, 0)),
                   pl.BlockSpec((BR, F2), lambda i: (i, 0))],
        out_shape=[jax.ShapeDtypeStruct((N, F2), jnp.float32),
                   jax.ShapeDtypeStruct((N, F2), jnp.float32)],
    )(acc, vec(bias1), vec(g1), vec(b1), vec(m1), vec(v1),
      wl2, bl2.reshape(1, F2), wr2, br2.reshape(1, F2))


def _final_body(acc_ref, bias_ref, g_ref, b_ref, m_ref, v_ref, wc_ref, bc_ref,
                o_ref):
    s = acc_ref[0] + acc_ref[1]
    h = s[:, 0:HID] / (s[:, HID:HID + 1] + 1e-16) + bias_ref[...]
    h = (h - m_ref[...]) * (g_ref[...] * lax.rsqrt(v_ref[...] + BN_EPS)) \
        + b_ref[...]
    h = jnp.where(h > 0, h, jnp.exp(h) - 1.0)
    o = jnp.dot(h, wc_ref[...], preferred_element_type=jnp.float32) \
        + bc_ref[...]
    o_ref[...] = 1.0 / (1.0 + jnp.exp(-o))


def _tc_final(acc, bias2, g2, b2, m2, v2, wc_pad, bc_pad):
    vec = lambda a: a.reshape(1, F2)
    return pl.pallas_call(
        _final_body,
        grid=(N // BR,),
        in_specs=[
            pl.BlockSpec((NC, BR, CW2), lambda i: (0, i, 0)),
            pl.BlockSpec((1, F2), lambda i: (0, 0)),
            pl.BlockSpec((1, F2), lambda i: (0, 0)),
            pl.BlockSpec((1, F2), lambda i: (0, 0)),
            pl.BlockSpec((1, F2), lambda i: (0, 0)),
            pl.BlockSpec((1, F2), lambda i: (0, 0)),
            pl.BlockSpec((F2, 128), lambda i: (0, 0)),
            pl.BlockSpec((1, 128), lambda i: (0, 0)),
        ],
        out_specs=pl.BlockSpec((BR, 128), lambda i: (i, 0)),
        out_shape=jax.ShapeDtypeStruct((N, 128), jnp.float32),
    )(acc, vec(bias2), vec(g2), vec(b2), vec(m2), vec(v2), wc_pad, bc_pad)


def kernel(x, edge_index, Wl1, bl1, Wr1, br1, att1, bias1, g1, b1, m1, v1,
           Wl2, bl2, Wr2, br2, att2, bias2, g2, b2, m2, v2, Wc, bc):
    loops = jnp.arange(N, dtype=jnp.int32)
    pad = jnp.zeros((E_PAD - ET,), jnp.int32)
    src = jnp.concatenate([edge_index[0].astype(jnp.int32), loops, pad])
    dst = jnp.concatenate([edge_index[1].astype(jnp.int32), loops, pad])

    xl1, xr1 = _tc_proj(x, Wl1, bl1, Wr1, br1)
    a0, a1, tmax1 = _alpha1(src, dst, xl1, xr1, att1.reshape(F1))
    acc1 = _agg1(src, dst, xl1, tmax1, a0, a1)[:, :N, :]
    xl2, xr2 = _tc_mid(acc1, bias1, g1, b1, m1, v1, Wl2, bl2, Wr2, br2)
    b0, tmax2 = _alpha2(src, dst, xl2, xr2, att2.reshape(F2))
    acc2 = _agg2(src, dst, xl2, tmax2, b0)[:, :N, :]
    wc_pad = jnp.pad(Wc, ((0, 0), (0, 127)))
    bc_pad = jnp.pad(bc, (0, 127)).reshape(1, 128)
    out = _tc_final(acc2, bias2, g2, b2, m2, v2, wc_pad, bc_pad)
    return out[:, :1]
